# baseline (speedup 1.0000x reference)
import functools
import jax
import jax.numpy as jnp
from jax import lax
from jax.experimental import pallas as pl
from jax.experimental.pallas import tpu as pltpu
from jax.experimental.pallas import tpu_sc as plsc

_INFO = plsc.get_sparse_core_info()
_NC, _NS = _INFO.num_cores, _INFO.num_subcores
_NW = _NC * _NS

_BATCH = 16384
_EMB_DIM = 64
_B_PER_W = _BATCH // _NW


@functools.partial(
    pl.kernel,
    mesh=plsc.VectorSubcoreMesh(core_axis_name="c", subcore_axis_name="s"),
    out_type=jax.ShapeDtypeStruct((_BATCH, _EMB_DIM), jnp.float32),
    scratch_types=[
        pltpu.VMEM((_B_PER_W,), jnp.int32),
        pltpu.VMEM((_B_PER_W, _EMB_DIM), jnp.float32),
        pltpu.SemaphoreType.DMA,
    ],
    compiler_params=pltpu.CompilerParams(use_tc_tiling_on_sc=False),
)
def _gather_kernel(idx_hbm, table_hbm, out_hbm, idx_v, rows_v, sem):
    wid = lax.axis_index("s") * _NC + lax.axis_index("c")
    base = wid * _B_PER_W
    pltpu.sync_copy(idx_hbm.at[pl.ds(base, _B_PER_W)], idx_v)
    pltpu.async_copy(table_hbm.at[idx_v], rows_v, sem).wait()
    pltpu.sync_copy(rows_v, out_hbm.at[pl.ds(base, _B_PER_W)])


def kernel(input, table):
    return _gather_kernel(input, table)


# speedup vs baseline: 1.0024x; 1.0024x over previous
import functools
import jax
import jax.numpy as jnp
from jax import lax
from jax.experimental import pallas as pl
from jax.experimental.pallas import tpu as pltpu
from jax.experimental.pallas import tpu_sc as plsc

_INFO = plsc.get_sparse_core_info()
_NC, _NS = _INFO.num_cores, _INFO.num_subcores
_NW = _NC * _NS

_BATCH = 16384
_EMB_DIM = 64
_B_PER_W = _BATCH // _NW
_N_CHUNKS = 8
_C = _B_PER_W // _N_CHUNKS


@functools.partial(
    pl.kernel,
    mesh=plsc.VectorSubcoreMesh(core_axis_name="c", subcore_axis_name="s"),
    out_type=jax.ShapeDtypeStruct((_BATCH, _EMB_DIM), jnp.float32),
    scratch_types=[
        pltpu.VMEM((_B_PER_W,), jnp.int32),
        pltpu.VMEM((_B_PER_W, _EMB_DIM), jnp.float32),
        pltpu.SemaphoreType.DMA,
        pltpu.SemaphoreType.DMA,
    ],
    compiler_params=pltpu.CompilerParams(use_tc_tiling_on_sc=False),
)
def _gather_kernel(idx_hbm, table_hbm, out_hbm, idx_v, rows_v, sem_g, sem_w):
    wid = lax.axis_index("s") * _NC + lax.axis_index("c")
    base = wid * _B_PER_W
    pltpu.sync_copy(idx_hbm.at[pl.ds(base, _B_PER_W)], idx_v)
    gathers = []
    for k in range(_N_CHUNKS):
        gathers.append(
            pltpu.async_copy(
                table_hbm.at[idx_v.at[pl.ds(k * _C, _C)]],
                rows_v.at[pl.ds(k * _C, _C)],
                sem_g,
            )
        )
    writes = []
    for k in range(_N_CHUNKS):
        gathers[k].wait()
        writes.append(
            pltpu.async_copy(
                rows_v.at[pl.ds(k * _C, _C)],
                out_hbm.at[pl.ds(base + k * _C, _C)],
                sem_w,
            )
        )
    for w in writes:
        w.wait()


def kernel(input, table):
    return _gather_kernel(input, table)


# speedup vs baseline: 1.7222x; 1.7181x over previous
import functools
import jax
import jax.numpy as jnp
from jax import lax
from jax.experimental import pallas as pl
from jax.experimental.pallas import tpu as pltpu
from jax.experimental.pallas import tpu_sc as plsc

_INFO = plsc.get_sparse_core_info()
_NC, _NS = _INFO.num_cores, _INFO.num_subcores
_NW = _NC * _NS

_BATCH = 16384
_EMB_DIM = 64
_B_PER_W = _BATCH // _NW


@functools.partial(
    pl.kernel,
    mesh=plsc.VectorSubcoreMesh(core_axis_name="c", subcore_axis_name="s"),
    out_type=jax.ShapeDtypeStruct((_BATCH, _EMB_DIM), jnp.float32),
    scratch_types=[
        pltpu.VMEM((_B_PER_W,), jnp.int32),
        pltpu.VMEM((_B_PER_W, _EMB_DIM), jnp.float32),
        pltpu.SemaphoreType.DMA,
        pltpu.SemaphoreType.DMA,
    ],
)
def _gather_kernel(idx_hbm, table_hbm, out_hbm, idx_v, rows_v, sem_i, sem_g):
    wid = lax.axis_index("s") * _NC + lax.axis_index("c")
    base = wid * _B_PER_W
    pltpu.async_copy(idx_hbm.at[pl.ds(base, _B_PER_W)], idx_v, sem_i).wait()

    @pl.loop(0, _B_PER_W // 16)
    def fire(k):
        vec = idx_v[pl.ds(k * 16, 16)]
        for l in range(16):
            row = vec[l]
            pltpu.async_copy(
                table_hbm.at[pl.ds(row, 1)],
                rows_v.at[pl.ds(k * 16 + l, 1)],
                sem_g,
            )

    @pl.loop(0, _B_PER_W, unroll=4)
    def drain(i):
        pltpu.make_async_copy(
            table_hbm.at[pl.ds(0, 1)], rows_v.at[pl.ds(0, 1)], sem_g
        ).wait()
    pltpu.sync_copy(rows_v, out_hbm.at[pl.ds(base, _B_PER_W)])


def kernel(input, table):
    return _gather_kernel(input, table)


# speedup vs baseline: 1.7260x; 1.0022x over previous
import functools
import jax
import jax.numpy as jnp
from jax import lax
from jax.experimental import pallas as pl
from jax.experimental.pallas import tpu as pltpu
from jax.experimental.pallas import tpu_sc as plsc

_INFO = plsc.get_sparse_core_info()
_NC, _NS = _INFO.num_cores, _INFO.num_subcores
_NW = _NC * _NS

_BATCH = 16384
_EMB_DIM = 64
_B_PER_W = _BATCH // _NW


@functools.partial(
    pl.kernel,
    mesh=plsc.VectorSubcoreMesh(core_axis_name="c", subcore_axis_name="s"),
    out_type=jax.ShapeDtypeStruct((_BATCH, _EMB_DIM), jnp.float32),
    scratch_types=[
        pltpu.VMEM((_B_PER_W,), jnp.int32),
        pltpu.VMEM((_B_PER_W, _EMB_DIM), jnp.float32),
        pltpu.SemaphoreType.DMA,
        pltpu.SemaphoreType.DMA,
    ],
)
def _gather_kernel(idx_hbm, table_hbm, out_hbm, idx_v, rows_v, sem_i, sem_g):
    wid = lax.axis_index("s") * _NC + lax.axis_index("c")
    base = wid * _B_PER_W
    pltpu.async_copy(idx_hbm.at[pl.ds(base, _B_PER_W)], idx_v, sem_i).wait()

    @plsc.parallel_loop(0, _B_PER_W // 16, unroll=2)
    def fire(k):
        vec = idx_v[pl.ds(k * 16, 16)]
        for l in range(16):
            row = vec[l]
            pltpu.async_copy(
                table_hbm.at[pl.ds(row, 1)],
                rows_v.at[pl.ds(k * 16 + l, 1)],
                sem_g,
            )

    pltpu.make_async_copy(
        table_hbm.at[pl.ds(0, _B_PER_W)], rows_v, sem_g
    ).wait()
    pltpu.sync_copy(rows_v, out_hbm.at[pl.ds(base, _B_PER_W)])


def kernel(input, table):
    return _gather_kernel(input, table)
